# baseline (speedup 1.0000x reference)
import functools

import jax
import jax.numpy as jnp
from jax import lax
from jax.experimental import pallas as pl
from jax.experimental.pallas import tpu as pltpu
from jax.experimental.pallas import tpu_sc as plsc

NUM_E = 64
N_TOK = 8192
D = 768
TM = 128
NBP = N_TOK // TM + NUM_E
M_PAD = NBP * TM

NC = 2
NS = 16
NW = NC * NS
ROWS_W = N_TOK // NW
CHUNK = 64
NCHUNK = ROWS_W // CHUNK


def _sc_mesh():
    return plsc.VectorSubcoreMesh(
        core_axis_name="c", subcore_axis_name="s",
        num_cores=NC, num_subcores=NS)


@functools.cache
def _dispatch_kernel():
    @functools.partial(
        pl.kernel,
        out_type=jax.ShapeDtypeStruct((M_PAD, D), jnp.float32),
        mesh=_sc_mesh(),
        scratch_types=[
            pltpu.VMEM((NCHUNK, CHUNK), jnp.int32),
            pltpu.VMEM((CHUNK, D), jnp.float32),
            pltpu.VMEM((CHUNK, D), jnp.float32),
            pltpu.SemaphoreType.DMA,
            pltpu.SemaphoreType.DMA,
            pltpu.SemaphoreType.DMA,
            pltpu.SemaphoreType.DMA,
        ],
    )
    def _dispatch_k(h_hbm, pp3_hbm, xs_hbm,
                    pp2, rows0, rows1, si0, si1, ss0, ss1):
        wid = lax.axis_index("s") * NC + lax.axis_index("c")
        base = wid * ROWS_W
        rows = (rows0, rows1)
        sin = (si0, si1)
        ssc = (ss0, ss1)
        pltpu.sync_copy(pp3_hbm.at[wid], pp2)

        def src(c):
            return h_hbm.at[pl.ds(base + c * CHUNK, CHUNK)]

        pltpu.async_copy(src(0), rows0, si0)
        pltpu.async_copy(src(1), rows1, si1)
        for c in range(NCHUNK):
            p = c % 2
            pltpu.make_async_copy(src(c), rows[p], sin[p]).wait()
            cp = pltpu.async_copy(rows[p], xs_hbm.at[pp2.at[c]], ssc[p])
            cp.wait()
            if c + 2 < NCHUNK:
                pltpu.async_copy(src(c + 2), rows[p], sin[p])

    return _dispatch_k


@functools.cache
def _undispatch_kernel():
    @functools.partial(
        pl.kernel,
        out_type=jax.ShapeDtypeStruct((N_TOK, D), jnp.float32),
        mesh=_sc_mesh(),
        scratch_types=[
            pltpu.VMEM((NCHUNK, CHUNK), jnp.int32),
            pltpu.VMEM((CHUNK, D), jnp.float32),
            pltpu.VMEM((CHUNK, D), jnp.float32),
            pltpu.SemaphoreType.DMA,
            pltpu.SemaphoreType.DMA,
            pltpu.SemaphoreType.DMA,
            pltpu.SemaphoreType.DMA,
        ],
    )
    def _undispatch_k(y_hbm, pp3_hbm, out_hbm,
                      pp2, rows0, rows1, sg0, sg1, sw0, sw1):
        wid = lax.axis_index("s") * NC + lax.axis_index("c")
        base = wid * ROWS_W
        rows = (rows0, rows1)
        sga = (sg0, sg1)
        swr = (sw0, sw1)
        pltpu.sync_copy(pp3_hbm.at[wid], pp2)
        pltpu.async_copy(y_hbm.at[pp2.at[0]], rows0, sg0)
        pltpu.async_copy(y_hbm.at[pp2.at[1]], rows1, sg1)
        for c in range(NCHUNK):
            p = c % 2
            pltpu.make_async_copy(y_hbm.at[pp2.at[c]], rows[p], sga[p]).wait()
            cp = pltpu.async_copy(rows[p], out_hbm.at[pl.ds(base + c * CHUNK, CHUNK)], swr[p])
            cp.wait()
            if c + 2 < NCHUNK:
                pltpu.async_copy(y_hbm.at[pp2.at[c + 2]], rows[p], sga[p])

    return _undispatch_k


def _mm_body(bexp_r, bxi_r, bxo_r, bval_r, x_r, w_r, b_r, rw_r, o_r):
    i = pl.program_id(0)

    @pl.when(bval_r[i] == 1)
    def _():
        acc = jnp.dot(x_r[...].astype(jnp.bfloat16),
                      w_r[0].astype(jnp.bfloat16),
                      preferred_element_type=jnp.float32)
        o_r[...] = (acc + b_r[0, 0]) * rw_r[...][:, :1]


def _grouped_matmul(xs_pad, W, b3, rw_pad, bexp, bxi, bxo, bval):
    grid_spec = pltpu.PrefetchScalarGridSpec(
        num_scalar_prefetch=4,
        grid=(NBP,),
        in_specs=[
            pl.BlockSpec((TM, D), lambda i, be, bx, bo, bv: (bx[i], 0)),
            pl.BlockSpec((1, D, D), lambda i, be, bx, bo, bv: (be[i], 0, 0)),
            pl.BlockSpec((1, 1, D), lambda i, be, bx, bo, bv: (be[i], 0, 0)),
            pl.BlockSpec((TM, 2), lambda i, be, bx, bo, bv: (bx[i], 0)),
        ],
        out_specs=pl.BlockSpec((TM, D), lambda i, be, bx, bo, bv: (bo[i], 0)),
    )
    return pl.pallas_call(
        _mm_body,
        grid_spec=grid_spec,
        out_shape=jax.ShapeDtypeStruct((M_PAD, D), jnp.float32),
        compiler_params=pltpu.CompilerParams(
            dimension_semantics=("parallel",)),
    )(bexp, bxi, bxo, bval, xs_pad, W, b3, rw_pad)


def _tables(primary):
    oh = (primary[:, None] == jnp.arange(NUM_E, dtype=jnp.int32)[None, :]
          ).astype(jnp.int16)
    run = jnp.cumsum(oh, axis=0)
    counts = run[-1].astype(jnp.int32)
    rank = jnp.sum(oh * run, axis=1).astype(jnp.int32) - 1
    nblk = (counts + TM - 1) // TM
    cum_nblk = jnp.cumsum(nblk)
    blk_off = cum_nblk - nblk
    used = cum_nblk[-1]
    base_blk = blk_off.astype(jnp.int16)
    pp = (jnp.sum(oh * base_blk[None, :], axis=1).astype(jnp.int32) * TM
          + rank)
    blocks = jnp.arange(NBP, dtype=jnp.int32)
    bexp_raw = jnp.searchsorted(cum_nblk, blocks, side="right").astype(jnp.int32)
    last = used - 1
    bval = (blocks < used).astype(jnp.int32)
    bexp = jnp.where(blocks < used, bexp_raw, bexp_raw[last]).astype(jnp.int32)
    bxi = jnp.where(blocks < used, blocks, last).astype(jnp.int32)
    bxo = jnp.where(blocks < used, blocks, NBP - 1).astype(jnp.int32)
    return pp, bexp, bxi, bxo, bval


def kernel(hidden_states, expert_indices, routing_weights, W, b):
    primary = expert_indices[:, 0].astype(jnp.int32)
    pp, bexp, bxi, bxo, bval = _tables(primary)
    pp3 = pp.reshape(NW, NCHUNK, CHUNK)
    xs_pad = _dispatch_kernel()(hidden_states, pp3)
    rw_pad = jnp.zeros((M_PAD, 2), jnp.float32).at[pp].set(routing_weights)
    y = _grouped_matmul(xs_pad, W, b[:, None, :], rw_pad, bexp, bxi, bxo, bval)
    return _undispatch_kernel()(y, pp3)


# speedup vs baseline: 1.0560x; 1.0560x over previous
import functools

import jax
import jax.numpy as jnp
from jax import lax
from jax.experimental import pallas as pl
from jax.experimental.pallas import tpu as pltpu
from jax.experimental.pallas import tpu_sc as plsc

NUM_E = 64
N_TOK = 8192
D = 768
TM = 256
NBP = N_TOK // TM + NUM_E
M_PAD = NBP * TM

NC = 2
NS = 16
NW = NC * NS
ROWS_W = N_TOK // NW
CHUNK = 64
NCHUNK = ROWS_W // CHUNK


def _sc_mesh():
    return plsc.VectorSubcoreMesh(
        core_axis_name="c", subcore_axis_name="s",
        num_cores=NC, num_subcores=NS)


@functools.cache
def _dispatch_kernel():
    @functools.partial(
        pl.kernel,
        out_type=jax.ShapeDtypeStruct((M_PAD, D), jnp.float32),
        mesh=_sc_mesh(),
        scratch_types=[
            pltpu.VMEM((NCHUNK, CHUNK), jnp.int32),
            pltpu.VMEM((CHUNK, D), jnp.float32),
            pltpu.VMEM((CHUNK, D), jnp.float32),
            pltpu.SemaphoreType.DMA,
            pltpu.SemaphoreType.DMA,
            pltpu.SemaphoreType.DMA,
            pltpu.SemaphoreType.DMA,
        ],
    )
    def _dispatch_k(h_hbm, pp3_hbm, xs_hbm,
                    pp2, rows0, rows1, si0, si1, ss0, ss1):
        wid = lax.axis_index("s") * NC + lax.axis_index("c")
        base = wid * ROWS_W
        rows = (rows0, rows1)
        sin = (si0, si1)
        ssc = (ss0, ss1)
        pltpu.sync_copy(pp3_hbm.at[wid], pp2)

        def src(c):
            return h_hbm.at[pl.ds(base + c * CHUNK, CHUNK)]

        pltpu.async_copy(src(0), rows0, si0)
        pltpu.async_copy(src(1), rows1, si1)
        for c in range(NCHUNK):
            p = c % 2
            pltpu.make_async_copy(src(c), rows[p], sin[p]).wait()
            cp = pltpu.async_copy(rows[p], xs_hbm.at[pp2.at[c]], ssc[p])
            cp.wait()
            if c + 2 < NCHUNK:
                pltpu.async_copy(src(c + 2), rows[p], sin[p])

    return _dispatch_k


@functools.cache
def _undispatch_kernel():
    @functools.partial(
        pl.kernel,
        out_type=jax.ShapeDtypeStruct((N_TOK, D), jnp.float32),
        mesh=_sc_mesh(),
        scratch_types=[
            pltpu.VMEM((NCHUNK, CHUNK), jnp.int32),
            pltpu.VMEM((CHUNK, D), jnp.float32),
            pltpu.VMEM((CHUNK, D), jnp.float32),
            pltpu.SemaphoreType.DMA,
            pltpu.SemaphoreType.DMA,
            pltpu.SemaphoreType.DMA,
            pltpu.SemaphoreType.DMA,
        ],
    )
    def _undispatch_k(y_hbm, pp3_hbm, out_hbm,
                      pp2, rows0, rows1, sg0, sg1, sw0, sw1):
        wid = lax.axis_index("s") * NC + lax.axis_index("c")
        base = wid * ROWS_W
        rows = (rows0, rows1)
        sga = (sg0, sg1)
        swr = (sw0, sw1)
        pltpu.sync_copy(pp3_hbm.at[wid], pp2)
        pltpu.async_copy(y_hbm.at[pp2.at[0]], rows0, sg0)
        pltpu.async_copy(y_hbm.at[pp2.at[1]], rows1, sg1)
        for c in range(NCHUNK):
            p = c % 2
            pltpu.make_async_copy(y_hbm.at[pp2.at[c]], rows[p], sga[p]).wait()
            cp = pltpu.async_copy(rows[p], out_hbm.at[pl.ds(base + c * CHUNK, CHUNK)], swr[p])
            cp.wait()
            if c + 2 < NCHUNK:
                pltpu.async_copy(y_hbm.at[pp2.at[c + 2]], rows[p], sga[p])

    return _undispatch_k


def _mm_body(bexp_r, bxi_r, bxo_r, bval_r, x_r, w_r, b_r, rw_r, o_r):
    i = pl.program_id(0)

    @pl.when(bval_r[i] == 1)
    def _():
        acc = jnp.dot(x_r[...].astype(jnp.bfloat16),
                      w_r[0].astype(jnp.bfloat16),
                      preferred_element_type=jnp.float32)
        o_r[...] = (acc + b_r[0, 0]) * rw_r[...][:, :1]


def _grouped_matmul(xs_pad, W, b3, rw_pad, bexp, bxi, bxo, bval):
    grid_spec = pltpu.PrefetchScalarGridSpec(
        num_scalar_prefetch=4,
        grid=(NBP,),
        in_specs=[
            pl.BlockSpec((TM, D), lambda i, be, bx, bo, bv: (bx[i], 0)),
            pl.BlockSpec((1, D, D), lambda i, be, bx, bo, bv: (be[i], 0, 0)),
            pl.BlockSpec((1, 1, D), lambda i, be, bx, bo, bv: (be[i], 0, 0)),
            pl.BlockSpec((TM, 2), lambda i, be, bx, bo, bv: (bx[i], 0)),
        ],
        out_specs=pl.BlockSpec((TM, D), lambda i, be, bx, bo, bv: (bo[i], 0)),
    )
    return pl.pallas_call(
        _mm_body,
        grid_spec=grid_spec,
        out_shape=jax.ShapeDtypeStruct((M_PAD, D), jnp.float32),
        compiler_params=pltpu.CompilerParams(
            dimension_semantics=("parallel",)),
    )(bexp, bxi, bxo, bval, xs_pad, W, b3, rw_pad)


def _tables(primary):
    oh = (primary[:, None] == jnp.arange(NUM_E, dtype=jnp.int32)[None, :]
          ).astype(jnp.int16)
    run = jnp.cumsum(oh, axis=0)
    counts = run[-1].astype(jnp.int32)
    rank = jnp.sum(oh * run, axis=1).astype(jnp.int32) - 1
    nblk = (counts + TM - 1) // TM
    cum_nblk = jnp.cumsum(nblk)
    blk_off = cum_nblk - nblk
    used = cum_nblk[-1]
    base_blk = blk_off.astype(jnp.int16)
    pp = (jnp.sum(oh * base_blk[None, :], axis=1).astype(jnp.int32) * TM
          + rank)
    blocks = jnp.arange(NBP, dtype=jnp.int32)
    bexp_raw = jnp.searchsorted(cum_nblk, blocks, side="right").astype(jnp.int32)
    last = used - 1
    bval = (blocks < used).astype(jnp.int32)
    bexp = jnp.where(blocks < used, bexp_raw, bexp_raw[last]).astype(jnp.int32)
    bxi = jnp.where(blocks < used, blocks, last).astype(jnp.int32)
    bxo = jnp.where(blocks < used, blocks, NBP - 1).astype(jnp.int32)
    return pp, bexp, bxi, bxo, bval


def kernel(hidden_states, expert_indices, routing_weights, W, b):
    primary = expert_indices[:, 0].astype(jnp.int32)
    pp, bexp, bxi, bxo, bval = _tables(primary)
    pp3 = pp.reshape(NW, NCHUNK, CHUNK)
    xs_pad = _dispatch_kernel()(hidden_states, pp3)
    rw_pad = jnp.zeros((M_PAD, 2), jnp.float32).at[pp].set(routing_weights)
    y = _grouped_matmul(xs_pad, W, b[:, None, :], rw_pad, bexp, bxi, bxo, bval)
    return _undispatch_kernel()(y, pp3)
